# baseline (speedup 1.0000x reference)
import functools
import math

import jax
import jax.numpy as jnp
from jax import lax
from jax.experimental import pallas as pl
from jax.experimental.pallas import tpu as pltpu
from jax.experimental.pallas import tpu_sc as plsc

D_MODEL = 64
SCALE = math.sqrt(D_MODEL)

NC, NS = 2, 16
NW = NC * NS
IDX_MINOR = 128
CHUNK_IR = 4
ROWS_PER_STAGE = IDX_MINOR * CHUNK_IR


@functools.cache
def _make_kernel(B: int):
    assert B % (NW * IDX_MINOR * CHUNK_IR) == 0
    ir_per_w = B // IDX_MINOR // NW
    stages = ir_per_w // CHUNK_IR
    mesh = plsc.VectorSubcoreMesh(core_axis_name="c", subcore_axis_name="s")

    @functools.partial(
        pl.kernel,
        out_type=jax.ShapeDtypeStruct((B, D_MODEL), jnp.float32),
        mesh=mesh,
        scratch_types=[
            pltpu.VMEM((ir_per_w, IDX_MINOR), jnp.int32),
            pltpu.VMEM((ROWS_PER_STAGE, D_MODEL), jnp.float32),
            pltpu.SemaphoreType.DMA,
        ],
        compiler_params=pltpu.CompilerParams(use_tc_tiling_on_sc=False),
    )
    def emb_kernel(idx_hbm, lut_hbm, out_hbm, idx_v, rows_v, sem):
        wid = lax.axis_index("s") * NC + lax.axis_index("c")
        ir_base = wid * ir_per_w
        pltpu.sync_copy(idx_hbm.at[pl.ds(ir_base, ir_per_w)], idx_v)

        def stage(g, carry):
            copies = [
                pltpu.async_copy(
                    lut_hbm.at[idx_v.at[g * CHUNK_IR + j]],
                    rows_v.at[pl.ds(j * IDX_MINOR, IDX_MINOR)],
                    sem,
                )
                for j in range(CHUNK_IR)
            ]
            for cp in copies:
                cp.wait()

            def scale_rows(r, c2):
                for c in range(D_MODEL // 16):
                    sl = pl.ds(c * 16, 16)
                    rows_v[r, sl] = rows_v[r, sl] * SCALE
                return c2

            lax.fori_loop(0, ROWS_PER_STAGE, scale_rows, 0)
            out_base = (ir_base + g * CHUNK_IR) * IDX_MINOR
            pltpu.sync_copy(rows_v, out_hbm.at[pl.ds(out_base, ROWS_PER_STAGE)])
            return carry

        lax.fori_loop(0, stages, stage, 0)

    return emb_kernel


def kernel(x, lut):
    B = x.shape[0] * x.shape[1]
    idx = x.reshape(B // IDX_MINOR, IDX_MINOR).astype(jnp.int32)
    out = _make_kernel(B)(idx, lut)
    return out.reshape(x.shape[0], x.shape[1], D_MODEL)
